# baseline (speedup 1.0000x reference)
import functools

import jax
import jax.numpy as jnp
from jax import lax
from jax.experimental import pallas as pl
from jax.experimental.pallas import tpu as pltpu
from jax.experimental.pallas import tpu_sc as plsc

N_ROWS = 100000
N_COLS = 128
NC = 2
NS = 16
L = 16
HALF = N_COLS // NC
GROUPS = HALF // L
CHUNK = 625
N_CHUNKS = N_ROWS // CHUNK
CHUNKS_PER_W = N_CHUNKS // NS

_mesh = plsc.VectorSubcoreMesh(core_axis_name="c", subcore_axis_name="s")


@functools.partial(
    pl.kernel,
    out_type=jax.ShapeDtypeStruct((1, N_COLS), jnp.float32),
    mesh=_mesh,
    scratch_types=[
        pltpu.VMEM((2, CHUNK, HALF), jnp.float32),
        pltpu.VMEM((NS, HALF), jnp.float32),
        pltpu.VMEM((HALF,), jnp.float32),
        pltpu.VMEM_SHARED((NS, HALF), jnp.float32),
        pltpu.SemaphoreType.DMA,
        pltpu.SemaphoreType.DMA,
    ],
    compiler_params=pltpu.CompilerParams(use_tc_tiling_on_sc=False),
)
def _readout_sc(x_hbm, out_hbm, bufs, red_v, acc_v, shared, sem0, sem1):
    c = lax.axis_index("c")
    s = lax.axis_index("s")
    col0 = c * HALF
    sems = (sem0, sem1)

    def src(j):
        row0 = (j * NS + s) * CHUNK
        return x_hbm.at[pl.ds(row0, CHUNK), pl.ds(col0, HALF)]

    accs = tuple(jnp.zeros((L,), jnp.float32) for _ in range(GROUPS))

    descs = [None, None]
    descs[0] = pltpu.async_copy(src(0), bufs.at[0], sems[0])
    for j in range(CHUNKS_PER_W):
        b = j % 2
        if j + 1 < CHUNKS_PER_W:
            descs[1 - b] = pltpu.async_copy(src(j + 1), bufs.at[1 - b], sems[1 - b])
        descs[b].wait()

        def body(r, a, _b=b):
            return tuple(a[g] + bufs[_b, r, g * L:(g + 1) * L]
                         for g in range(GROUPS))

        accs = lax.fori_loop(0, CHUNK, body, accs)

    for g in range(GROUPS):
        acc_v[pl.ds(g * L, L)] = accs[g]
    pltpu.sync_copy(acc_v, shared.at[s])
    plsc.subcore_barrier()

    @pl.when(s == 0)
    def _():
        pltpu.sync_copy(shared, red_v)

        def body2(i, a):
            return tuple(a[g] + red_v[i, g * L:(g + 1) * L]
                         for g in range(GROUPS))

        faccs = lax.fori_loop(
            0, NS, body2,
            tuple(jnp.zeros((L,), jnp.float32) for _ in range(GROUPS)))
        for g in range(GROUPS):
            acc_v[pl.ds(g * L, L)] = faccs[g]
        pltpu.sync_copy(acc_v, out_hbm.at[0, pl.ds(col0, HALF)])


def kernel(x):
    return _readout_sc(x)


# speedup vs baseline: 1.0620x; 1.0620x over previous
import functools

import jax
import jax.numpy as jnp
from jax import lax
from jax.experimental import pallas as pl
from jax.experimental.pallas import tpu as pltpu
from jax.experimental.pallas import tpu_sc as plsc

N_ROWS = 100000
N_COLS = 128
NC = 2
NS = 16
L = 16
HALF = N_COLS // NC
GROUPS = HALF // L
CHUNK = 625
N_CHUNKS = N_ROWS // CHUNK
CHUNKS_PER_W = N_CHUNKS // NS

_mesh = plsc.VectorSubcoreMesh(core_axis_name="c", subcore_axis_name="s")


@functools.partial(
    pl.kernel,
    out_type=jax.ShapeDtypeStruct((1, N_COLS), jnp.float32),
    mesh=_mesh,
    scratch_types=[
        pltpu.VMEM((2, CHUNK, HALF), jnp.float32),
        pltpu.VMEM((NS, HALF), jnp.float32),
        pltpu.VMEM((HALF,), jnp.float32),
        pltpu.VMEM_SHARED((NS, HALF), jnp.float32),
        pltpu.SemaphoreType.DMA,
        pltpu.SemaphoreType.DMA,
    ],
    compiler_params=pltpu.CompilerParams(use_tc_tiling_on_sc=False),
)
def _readout_sc(x_hbm, out_hbm, bufs, red_v, acc_v, shared, sem0, sem1):
    c = lax.axis_index("c")
    s = lax.axis_index("s")
    col0 = c * HALF
    sems = (sem0, sem1)

    def src(j):
        row0 = (j * NS + s) * CHUNK
        return x_hbm.at[pl.ds(row0, CHUNK), pl.ds(col0, HALF)]

    accs = tuple(jnp.zeros((L,), jnp.float32) for _ in range(2 * GROUPS))

    descs = [None, None]
    descs[0] = pltpu.async_copy(src(0), bufs.at[0], sems[0])
    for j in range(CHUNKS_PER_W):
        b = j % 2
        if j + 1 < CHUNKS_PER_W:
            descs[1 - b] = pltpu.async_copy(src(j + 1), bufs.at[1 - b], sems[1 - b])
        descs[b].wait()

        def body(r, a, _b=b):
            lo = tuple(a[g] + bufs[_b, r, g * L:(g + 1) * L]
                       for g in range(GROUPS))
            hi = tuple(a[GROUPS + g] + bufs[_b, r + 1, g * L:(g + 1) * L]
                       for g in range(GROUPS))
            return lo + hi

        accs = plsc.parallel_loop(0, CHUNK, step=2, unroll=4, carry=accs)(body)

    accs = tuple(accs[g] + accs[GROUPS + g] for g in range(GROUPS))

    for g in range(GROUPS):
        acc_v[pl.ds(g * L, L)] = accs[g]
    pltpu.sync_copy(acc_v, shared.at[s])
    plsc.subcore_barrier()

    @pl.when(s == 0)
    def _():
        pltpu.sync_copy(shared, red_v)

        def body2(i, a):
            return tuple(a[g] + red_v[i, g * L:(g + 1) * L]
                         for g in range(GROUPS))

        faccs = lax.fori_loop(
            0, NS, body2,
            tuple(jnp.zeros((L,), jnp.float32) for _ in range(GROUPS)))
        for g in range(GROUPS):
            acc_v[pl.ds(g * L, L)] = faccs[g]
        pltpu.sync_copy(acc_v, out_hbm.at[0, pl.ds(col0, HALF)])


def kernel(x):
    return _readout_sc(x)


# speedup vs baseline: 1.0654x; 1.0032x over previous
import functools

import jax
import jax.numpy as jnp
from jax import lax
from jax.experimental import pallas as pl
from jax.experimental.pallas import tpu as pltpu
from jax.experimental.pallas import tpu_sc as plsc

N_ROWS = 100000
N_COLS = 128
NC = 2
NS = 16
L = 16
HALF = N_COLS // NC
GROUPS = HALF // L
CHUNK = 625
N_CHUNKS = N_ROWS // CHUNK
CHUNKS_PER_W = N_CHUNKS // NS

_mesh = plsc.VectorSubcoreMesh(core_axis_name="c", subcore_axis_name="s")


@functools.partial(
    pl.kernel,
    out_type=jax.ShapeDtypeStruct((1, N_COLS), jnp.float32),
    mesh=_mesh,
    scratch_types=[
        pltpu.VMEM((2, CHUNK, HALF), jnp.float32),
        pltpu.VMEM((NS, HALF), jnp.float32),
        pltpu.VMEM((HALF,), jnp.float32),
        pltpu.VMEM_SHARED((NS, HALF), jnp.float32),
        pltpu.SemaphoreType.DMA,
        pltpu.SemaphoreType.DMA,
    ],
    compiler_params=pltpu.CompilerParams(use_tc_tiling_on_sc=False),
)
def _readout_sc(x_hbm, out_hbm, bufs, red_v, acc_v, shared, sem0, sem1):
    c = lax.axis_index("c")
    s = lax.axis_index("s")
    col0 = c * HALF
    sems = (sem0, sem1)

    def src(j):
        row0 = (j * NS + s) * CHUNK
        return x_hbm.at[pl.ds(row0, CHUNK), pl.ds(col0, HALF)]

    accs = tuple(jnp.zeros((L,), jnp.float32) for _ in range(2 * GROUPS))

    descs = [None, None]
    descs[0] = pltpu.async_copy(src(0), bufs.at[0], sems[0])
    for j in range(CHUNKS_PER_W):
        b = j % 2
        if j + 1 < CHUNKS_PER_W:
            descs[1 - b] = pltpu.async_copy(src(j + 1), bufs.at[1 - b], sems[1 - b])
        descs[b].wait()

        def body(r, a, _b=b):
            lo = tuple(a[g] + bufs[_b, r, g * L:(g + 1) * L]
                       for g in range(GROUPS))
            hi = tuple(a[GROUPS + g] + bufs[_b, r + 1, g * L:(g + 1) * L]
                       for g in range(GROUPS))
            return lo + hi

        accs = plsc.parallel_loop(0, CHUNK - 1, step=2, unroll=4, carry=accs)(body)
        accs = (tuple(accs[g] + bufs[b, CHUNK - 1, g * L:(g + 1) * L]
                      for g in range(GROUPS))
                + tuple(accs[GROUPS + g] for g in range(GROUPS)))

    accs = tuple(accs[g] + accs[GROUPS + g] for g in range(GROUPS))

    for g in range(GROUPS):
        acc_v[pl.ds(g * L, L)] = accs[g]
    pltpu.sync_copy(acc_v, shared.at[s])
    plsc.subcore_barrier()

    @pl.when(s == 0)
    def _():
        pltpu.sync_copy(shared, red_v)

        def body2(i, a):
            return tuple(a[g] + red_v[i, g * L:(g + 1) * L]
                         for g in range(GROUPS))

        faccs = lax.fori_loop(
            0, NS, body2,
            tuple(jnp.zeros((L,), jnp.float32) for _ in range(GROUPS)))
        for g in range(GROUPS):
            acc_v[pl.ds(g * L, L)] = faccs[g]
        pltpu.sync_copy(acc_v, out_hbm.at[0, pl.ds(col0, HALF)])


def kernel(x):
    return _readout_sc(x)
